# baseline (speedup 1.0000x reference)
import functools

import jax
import jax.numpy as jnp
from jax import lax
from jax.experimental import pallas as pl
from jax.experimental.pallas import tpu as pltpu
from jax.experimental.pallas import tpu_sc as plsc

NODES = 10000
DIM = 128
OUT = 128

NC = 2
NS = 16
NW = NC * NS
CH = 80
NCH = 500
SUP = 5
NSUP = NCH // SUP
NPAR = 4
GSTEP = NPAR * SUP
ZROWS = 632
ZLAST = NODES - 15 * ZROWS


def _sc_segment_sum(h, src, dst, zeros):
    mesh = plsc.VectorSubcoreMesh(core_axis_name="c", subcore_axis_name="s")
    NBUF = 4
    SKEW = 2

    @functools.partial(
        pl.kernel,
        mesh=mesh,
        out_type=jax.ShapeDtypeStruct((NC, NODES, OUT), jnp.float32),
        scratch_types=[
            pltpu.VMEM_SHARED((NODES, OUT), jnp.float32),
            pltpu.VMEM((NPAR, SUP, CH), jnp.int32),
            pltpu.VMEM((NPAR, SUP, CH), jnp.int32),
            pltpu.VMEM((NBUF, CH, OUT), jnp.float32),
            pltpu.SemaphoreType.DMA,
            pltpu.SemaphoreType.DMA,
            pltpu.SemaphoreType.DMA,
            pltpu.SemaphoreType.DMA,
            pltpu.SemaphoreType.DMA,
            pltpu.SemaphoreType.DMA,
            pltpu.SemaphoreType.DMA,
            pltpu.SemaphoreType.DMA,
            pltpu.SemaphoreType.DMA,
            pltpu.SemaphoreType.DMA,
            pltpu.SemaphoreType.DMA,
            pltpu.SemaphoreType.DMA,
        ],
    )
    def k(h_hbm, src_hbm, dst_hbm, zeros_hbm, out_hbm,
          acc, src_v, dst_v, rows_v, *sems12):
        semg = list(sems12[:NBUF])
        sems = list(sems12[NBUF:2 * NBUF])
        semi = list(sems12[2 * NBUF:])
        cid = lax.axis_index("c")
        sid = lax.axis_index("s")
        wid = cid * NS + sid

        def load_idx(sp, par):
            pltpu.async_copy(src_hbm.at[wid, sp], src_v.at[par], semi[par])
            pltpu.async_copy(dst_hbm.at[wid, sp], dst_v.at[par], semi[par])

        def wait_idx(par):
            pltpu.make_async_copy(src_hbm.at[0, 0], src_v.at[0],
                                  semi[par]).wait()
            pltpu.make_async_copy(dst_hbm.at[0, 0], dst_v.at[0],
                                  semi[par]).wait()

        def gather(par, c, b):
            pltpu.async_copy(h_hbm.at[src_v.at[par, c]], rows_v.at[b],
                             semg[b])

        def wait_gather(b):
            pltpu.make_async_copy(h_hbm.at[src_v.at[0, 0]], rows_v.at[b],
                                  semg[b]).wait()

        def scatter(par, c, b):
            pltpu.async_copy(rows_v.at[b], acc.at[dst_v.at[par, c]],
                             sems[b], add=True)

        def wait_scatter(b):
            pltpu.make_async_copy(rows_v.at[b], acc.at[dst_v.at[0, 0]],
                                  sems[b]).wait()

        def rows_slice(ref):
            return ref.at[pl.ds(sid * ZROWS, ZROWS)]

        def rows_slice_last(ref):
            return ref.at[pl.ds(15 * ZROWS, ZLAST)]

        load_idx(0, 0)
        load_idx(1, 1)
        load_idx(2, 2)
        wait_idx(0)
        for b in range(SKEW):
            gather(0, b, b)
        @pl.when(sid < 15)
        def _():
            pltpu.sync_copy(zeros_hbm, rows_slice(acc))

        @pl.when(sid == 15)
        def _():
            pltpu.sync_copy(zeros_hbm.at[pl.ds(0, ZLAST)],
                            rows_slice_last(acc))

        plsc.subcore_barrier()

        def group(G, carry):
            for j in range(GSTEP):
                lane, c = j // SUP, j % SUP
                b = j % NBUF
                tg = G * GSTEP + j
                if j % SUP == 0:
                    sp_load = 4 * G + 3 + lane
                    par_load = (3 + lane) % NPAR

                    @pl.when(sp_load < NSUP)
                    def _():
                        load_idx(sp_load, par_load)

                wait_gather(b)
                scatter(lane, c, b)
                bn = (b + SKEW) % NBUF

                @pl.when(tg >= SKEW)
                def _():
                    wait_scatter(bn)

                if (j + SKEW) % SUP == 0:
                    sp_use = 4 * G + (j + SKEW) // SUP
                    par_use = ((j + SKEW) // SUP) % NPAR

                    @pl.when(sp_use < NSUP)
                    def _():
                        wait_idx(par_use)

                jn = j + SKEW
                par2 = (jn // SUP) % NPAR
                c2 = jn % SUP

                @pl.when(tg + SKEW < NCH)
                def _():
                    gather(par2, c2, bn)
            return carry

        lax.fori_loop(0, NCH // GSTEP, group, 0)
        for tg in range(NCH - SKEW, NCH):
            wait_scatter(tg % NBUF)
        plsc.subcore_barrier()

        @pl.when(sid < 15)
        def _():
            pltpu.sync_copy(rows_slice(acc),
                            out_hbm.at[cid, pl.ds(sid * ZROWS, ZROWS)])

        @pl.when(sid == 15)
        def _():
            pltpu.sync_copy(rows_slice_last(acc),
                            out_hbm.at[cid, pl.ds(15 * ZROWS, ZLAST)])

    return k(h, src, dst, zeros)


def _tc_matmul_body(x_ref, w_ref, o_ref):
    o_ref[...] = jnp.dot(x_ref[...], w_ref[...],
                         preferred_element_type=jnp.float32)


def _tc_matmul(x, W):
    BM = 2000
    return pl.pallas_call(
        _tc_matmul_body,
        grid=(NODES // BM,),
        in_specs=[
            pl.BlockSpec((BM, DIM), lambda i: (i, 0)),
            pl.BlockSpec((DIM, OUT), lambda i: (0, 0)),
        ],
        out_specs=pl.BlockSpec((BM, OUT), lambda i: (i, 0)),
        out_shape=jax.ShapeDtypeStruct((NODES, OUT), jnp.float32),
    )(x, W)


def _tc_finish_body(h1_ref, p_ref, v_ref, o_ref):
    s = p_ref[0] + p_ref[1]
    sv = jnp.dot(s, v_ref[...], preferred_element_type=jnp.float32)
    o_ref[...] = jnp.maximum(h1_ref[...] + sv, 0.0)


def _tc_finish(h1, partials, V):
    BM = 2000
    return pl.pallas_call(
        _tc_finish_body,
        grid=(NODES // BM,),
        in_specs=[
            pl.BlockSpec((BM, OUT), lambda i: (i, 0)),
            pl.BlockSpec((NC, BM, OUT), lambda i: (0, i, 0)),
            pl.BlockSpec((OUT, OUT), lambda i: (0, 0)),
        ],
        out_specs=pl.BlockSpec((BM, OUT), lambda i: (i, 0)),
        out_shape=jax.ShapeDtypeStruct((NODES, OUT), jnp.float32),
    )(h1, partials, V)


def kernel(x, edge_index, h, W, V, alpha):
    del alpha
    src = edge_index[:, 0, :].reshape(NW, NSUP, SUP, CH).astype(jnp.int32)
    dst = edge_index[:, 1, :].reshape(NW, NSUP, SUP, CH).astype(jnp.int32)
    zeros = jnp.zeros((ZROWS, OUT), jnp.float32)
    h1 = _tc_matmul(x, W)
    partials = _sc_segment_sum(h, src, dst, zeros)
    return _tc_finish(h1, partials, V)


# speedup vs baseline: 1.0032x; 1.0032x over previous
import functools

import jax
import jax.numpy as jnp
from jax import lax
from jax.experimental import pallas as pl
from jax.experimental.pallas import tpu as pltpu
from jax.experimental.pallas import tpu_sc as plsc

NODES = 10000
DIM = 128
OUT = 128

NC = 2
NS = 16
NW = NC * NS
CH = 80
NCH = 500
SUP = 5
NSUP = NCH // SUP
NPAR = 4
GSTEP = NPAR * SUP
ZROWS = 632
ZLAST = NODES - 15 * ZROWS


def _sc_segment_sum(h, src, dst, zeros):
    mesh = plsc.VectorSubcoreMesh(core_axis_name="c", subcore_axis_name="s")
    NBUF = 4
    SKEW = 2

    @functools.partial(
        pl.kernel,
        mesh=mesh,
        out_type=jax.ShapeDtypeStruct((NC, NODES, OUT), jnp.float32),
        scratch_types=[
            pltpu.VMEM_SHARED((NODES, OUT), jnp.float32),
            pltpu.VMEM((NPAR, SUP, CH), jnp.int32),
            pltpu.VMEM((NPAR, SUP, CH), jnp.int32),
            pltpu.VMEM((NBUF, CH, OUT), jnp.float32),
            pltpu.SemaphoreType.DMA,
            pltpu.SemaphoreType.DMA,
            pltpu.SemaphoreType.DMA,
            pltpu.SemaphoreType.DMA,
            pltpu.SemaphoreType.DMA,
            pltpu.SemaphoreType.DMA,
            pltpu.SemaphoreType.DMA,
            pltpu.SemaphoreType.DMA,
            pltpu.SemaphoreType.DMA,
            pltpu.SemaphoreType.DMA,
            pltpu.SemaphoreType.DMA,
            pltpu.SemaphoreType.DMA,
        ],
    )
    def k(h_hbm, src_hbm, dst_hbm, zeros_hbm, out_hbm,
          acc, src_v, dst_v, rows_v, *sems12):
        semg = list(sems12[:NBUF])
        sems = list(sems12[NBUF:2 * NBUF])
        semi = list(sems12[2 * NBUF:])
        cid = lax.axis_index("c")
        sid = lax.axis_index("s")
        wid = cid * NS + sid

        def load_idx(sp, par):
            pltpu.async_copy(src_hbm.at[wid, sp], src_v.at[par], semi[par])
            pltpu.async_copy(dst_hbm.at[wid, sp], dst_v.at[par], semi[par])

        def wait_idx(par):
            pltpu.make_async_copy(src_hbm.at[0, 0], src_v.at[0],
                                  semi[par]).wait()
            pltpu.make_async_copy(dst_hbm.at[0, 0], dst_v.at[0],
                                  semi[par]).wait()

        def gather(par, c, b):
            pltpu.async_copy(h_hbm.at[src_v.at[par, c]], rows_v.at[b],
                             semg[b])

        def wait_gather(b):
            pltpu.make_async_copy(h_hbm.at[src_v.at[0, 0]], rows_v.at[b],
                                  semg[b]).wait()

        def scatter(par, c, b):
            pltpu.async_copy(rows_v.at[b], acc.at[dst_v.at[par, c]],
                             sems[b], add=True)

        def wait_scatter(b):
            pltpu.make_async_copy(rows_v.at[b], acc.at[dst_v.at[0, 0]],
                                  sems[b]).wait()

        def rows_slice(ref):
            return ref.at[pl.ds(sid * ZROWS, ZROWS)]

        def rows_slice_last(ref):
            return ref.at[pl.ds(15 * ZROWS, ZLAST)]

        load_idx(0, 0)
        load_idx(1, 1)
        load_idx(2, 2)
        wait_idx(0)
        for b in range(SKEW):
            gather(0, b, b)
        @pl.when(sid < 15)
        def _():
            pltpu.sync_copy(zeros_hbm, rows_slice(acc))

        @pl.when(sid == 15)
        def _():
            pltpu.sync_copy(zeros_hbm.at[pl.ds(0, ZLAST)],
                            rows_slice_last(acc))

        plsc.subcore_barrier()

        def group(G, carry):
            for j in range(GSTEP):
                lane, c = j // SUP, j % SUP
                b = j % NBUF
                tg = G * GSTEP + j
                if j % SUP == 0:
                    sp_load = 4 * G + 3 + lane
                    par_load = (3 + lane) % NPAR

                    @pl.when(sp_load < NSUP)
                    def _():
                        load_idx(sp_load, par_load)

                wait_gather(b)
                scatter(lane, c, b)
                bn = (b + SKEW) % NBUF

                @pl.when(tg >= SKEW)
                def _():
                    wait_scatter(bn)

                if (j + SKEW) % SUP == 0:
                    sp_use = 4 * G + (j + SKEW) // SUP
                    par_use = ((j + SKEW) // SUP) % NPAR

                    @pl.when(sp_use < NSUP)
                    def _():
                        wait_idx(par_use)

                jn = j + SKEW
                par2 = (jn // SUP) % NPAR
                c2 = jn % SUP

                @pl.when(tg + SKEW < NCH)
                def _():
                    gather(par2, c2, bn)
            return carry

        lax.fori_loop(0, NCH // GSTEP, group, 0)
        for tg in range(NCH - SKEW, NCH):
            wait_scatter(tg % NBUF)
        plsc.subcore_barrier()

        @pl.when(sid < 15)
        def _():
            pltpu.sync_copy(rows_slice(acc),
                            out_hbm.at[cid, pl.ds(sid * ZROWS, ZROWS)])

        @pl.when(sid == 15)
        def _():
            pltpu.sync_copy(rows_slice_last(acc),
                            out_hbm.at[cid, pl.ds(15 * ZROWS, ZLAST)])

    return k(h, src, dst, zeros)


def _tc_finish_body(x_ref, w_ref, p_ref, v_ref, o_ref):
    xw = jnp.dot(x_ref[...], w_ref[...], preferred_element_type=jnp.float32)
    s = p_ref[0] + p_ref[1]
    sv = jnp.dot(s, v_ref[...], preferred_element_type=jnp.float32)
    o_ref[...] = jnp.maximum(xw + sv, 0.0)


def _tc_finish(x, W, partials, V):
    BM = 2000
    return pl.pallas_call(
        _tc_finish_body,
        grid=(NODES // BM,),
        in_specs=[
            pl.BlockSpec((BM, DIM), lambda i: (i, 0)),
            pl.BlockSpec((DIM, OUT), lambda i: (0, 0)),
            pl.BlockSpec((NC, BM, OUT), lambda i: (0, i, 0)),
            pl.BlockSpec((OUT, OUT), lambda i: (0, 0)),
        ],
        out_specs=pl.BlockSpec((BM, OUT), lambda i: (i, 0)),
        out_shape=jax.ShapeDtypeStruct((NODES, OUT), jnp.float32),
    )(x, W, partials, V)


def kernel(x, edge_index, h, W, V, alpha):
    del alpha
    src = edge_index[:, 0, :].reshape(NW, NSUP, SUP, CH).astype(jnp.int32)
    dst = edge_index[:, 1, :].reshape(NW, NSUP, SUP, CH).astype(jnp.int32)
    zeros = jnp.zeros((ZROWS, OUT), jnp.float32)
    partials = _sc_segment_sum(h, src, dst, zeros)
    return _tc_finish(x, W, partials, V)
